# baseline (speedup 1.0000x reference)
import jax
import jax.numpy as jnp
from jax import lax
from jax.experimental import pallas as pl
from jax.experimental.pallas import tpu as pltpu
from jax.experimental.pallas import tpu_sc as plsc

VOCAB = 1000000
D = 64
DPAD = 128
B = 4096
L = 50

NC = 2
NS = 16
NW = NC * NS
BPW = B // NW
LANES = 16
DREG = D // LANES


def _sc_body(idx_hbm, table_hbm, out_hbm, idx_v, acc_v, out_v, sem):
    wid = lax.axis_index("s") * NC + lax.axis_index("c")
    base = wid * BPW

    def stage(l, carry):
        pltpu.sync_copy(idx_hbm.at[pl.ds(l * B + base, BPW)], idx_v.at[l])
        return carry

    lax.fori_loop(0, L, stage, 0)

    zero = jnp.zeros((LANES,), jnp.float32)

    def zero_row(r, carry):
        for j in range(DPAD // LANES):
            acc_v[r, pl.ds(j * LANES, LANES)] = zero
        return carry

    lax.fori_loop(0, BPW, zero_row, 0)

    def fire(l, carry):
        pltpu.async_copy(table_hbm.at[idx_v.at[l]], acc_v, sem, add=True)
        return carry

    lax.fori_loop(0, L, fire, 0)

    def drain(l, carry):
        pltpu.make_async_copy(table_hbm.at[idx_v.at[0]], acc_v, sem).wait()
        return carry

    lax.fori_loop(0, L, drain, 0)

    scale = jnp.full((LANES,), 1.0 / L, jnp.float32)

    def scale_row(r, carry):
        for j in range(DREG):
            sl = pl.ds(j * LANES, LANES)
            out_v[r, sl] = acc_v[r, sl] * scale
        return carry

    lax.fori_loop(0, BPW, scale_row, 0)

    pltpu.sync_copy(out_v, out_hbm.at[pl.ds(base, BPW)])


def _encode(idx_lmajor, table_pad):
    mesh = plsc.VectorSubcoreMesh(core_axis_name="c", subcore_axis_name="s")
    return pl.kernel(
        _sc_body,
        out_type=jax.ShapeDtypeStruct((B, D), jnp.float32),
        mesh=mesh,
        scratch_types=[
            pltpu.VMEM((L, BPW), jnp.int32),
            pltpu.VMEM((BPW, DPAD), jnp.float32),
            pltpu.VMEM((BPW, D), jnp.float32),
            pltpu.SemaphoreType.DMA,
        ],
        compiler_params=pltpu.CompilerParams(use_tc_tiling_on_sc=False),
    )(idx_lmajor, table_pad)


@jax.jit
def kernel(sentence, table):
    idx_lmajor = jnp.transpose(sentence[0]).astype(jnp.int32).reshape(L * B)
    table_pad = jnp.concatenate([table, table], axis=1)
    return _encode(idx_lmajor, table_pad)


# speedup vs baseline: 1.2890x; 1.2890x over previous
import jax
import jax.numpy as jnp
from jax import lax
from jax.experimental import pallas as pl
from jax.experimental.pallas import tpu as pltpu
from jax.experimental.pallas import tpu_sc as plsc

VOCAB = 1000000
D = 64
DPAD = 128
B = 4096
L = 50

NC = 2
NS = 16
NW = NC * NS
BPW = B // NW
LANES = 16
DREG = D // LANES


def _sc_body(idx_hbm, table_hbm, out_hbm, idx_v, acc_v, sem):
    wid = lax.axis_index("s") * NC + lax.axis_index("c")
    base = wid * BPW

    def stage(l, carry):
        pltpu.sync_copy(idx_hbm.at[pl.ds(l * B + base, BPW)], idx_v.at[l])
        return carry

    lax.fori_loop(0, L, stage, 0)

    zero = jnp.zeros((LANES,), jnp.float32)

    def zero_row(r, carry):
        for j in range(DREG):
            acc_v[r, pl.ds(j * LANES, LANES)] = zero
        return carry

    lax.fori_loop(0, BPW, zero_row, 0)

    def fire(l, carry):
        pltpu.async_copy(table_hbm.at[idx_v.at[l]], acc_v, sem, add=True)
        return carry

    lax.fori_loop(0, L, fire, 0)

    def drain(l, carry):
        pltpu.make_async_copy(table_hbm.at[idx_v.at[0]], acc_v, sem).wait()
        return carry

    lax.fori_loop(0, L, drain, 0)

    scale = jnp.full((LANES,), 1.0 / L, jnp.float32)

    def scale_row(r, carry):
        for j in range(DREG):
            sl = pl.ds(j * LANES, LANES)
            acc_v[r, sl] = acc_v[r, sl] * scale
        return carry

    lax.fori_loop(0, BPW, scale_row, 0)

    pltpu.sync_copy(acc_v, out_hbm.at[pl.ds(base, BPW)])


def _encode(idx_lmajor, table_pad):
    mesh = plsc.VectorSubcoreMesh(core_axis_name="c", subcore_axis_name="s")
    return pl.kernel(
        _sc_body,
        out_type=jax.ShapeDtypeStruct((B, D), jnp.float32),
        mesh=mesh,
        scratch_types=[
            pltpu.VMEM((L, BPW), jnp.int32),
            pltpu.VMEM((BPW, D), jnp.float32),
            pltpu.SemaphoreType.DMA,
        ],
        compiler_params=pltpu.CompilerParams(use_tc_tiling_on_sc=False),
    )(idx_lmajor, table_pad)


@jax.jit
def kernel(sentence, table):
    idx_lmajor = jnp.transpose(sentence[0]).astype(jnp.int32).reshape(L * B) * 2
    table_pad = jnp.pad(table, ((0, 0), (0, DPAD - D)))
    table_view = jnp.reshape(table_pad, (2 * VOCAB, D))
    return _encode(idx_lmajor, table_view)
